# baseline (speedup 1.0000x reference)
import functools

import jax
import jax.numpy as jnp
from jax import lax
from jax.experimental import pallas as pl
from jax.experimental.pallas import tpu as pltpu
from jax.experimental.pallas import tpu_sc as plsc

EMBED_DIM = 32
BATCH = 16384

_info = plsc.get_sparse_core_info()
_NC, _NS = _info.num_cores, _info.num_subcores
_NW = _NC * _NS


def _make_gather(V, D, B):
    b_per_w = B // _NW
    mesh = plsc.VectorSubcoreMesh(core_axis_name="c", subcore_axis_name="s")

    @functools.partial(
        pl.kernel,
        mesh=mesh,
        out_type=jax.ShapeDtypeStruct((B, D), jnp.float32),
        scratch_types=[
            pltpu.VMEM((b_per_w,), jnp.int32),
            pltpu.SemaphoreType.DMA,
        ],
    )
    def gather_kernel(idx_hbm, table_hbm, out_hbm, idx_v, sem):
        wid = lax.axis_index("s") * _NC + lax.axis_index("c")
        base = wid * b_per_w

        pltpu.sync_copy(idx_hbm.at[pl.ds(base, b_per_w)], idx_v)

        unroll = 16

        def fire(g, _):
            i0 = g * unroll
            rows = idx_v[pl.ds(i0, unroll)]
            for j in range(unroll):
                pltpu.async_copy(
                    table_hbm.at[pl.ds(rows[j], 1)],
                    out_hbm.at[pl.ds(base + i0 + j, 1)],
                    sem,
                )
            return 0

        lax.fori_loop(0, b_per_w // unroll, fire, 0, unroll=1)

        def drain(g, _):
            for _j in range(unroll):
                pltpu.make_async_copy(
                    table_hbm.at[pl.ds(0, 1)],
                    out_hbm.at[pl.ds(base, 1)],
                    sem,
                ).wait()
            return 0

        lax.fori_loop(0, b_per_w // unroll, drain, 0, unroll=1)

    return gather_kernel


_gather = _make_gather(1_000_000, EMBED_DIM, BATCH)


@jax.jit
def kernel(indices, weight):
    return _gather(indices.astype(jnp.int32), weight)
